# baseline (speedup 1.0000x reference)
import jax
import jax.numpy as jnp
from jax.experimental import pallas as pl
from jax.experimental.pallas import tpu as pltpu

_T = 0.05
_BN = 2176
_NPAD = 100096
_CW = 50048


def _matmul_kernel(x_ref, memt_ref, out_ref):
    acc = jax.lax.dot_general(
        x_ref[...],
        memt_ref[...],
        dimension_numbers=(((1,), (0,)), ((), ())),
        preferred_element_type=jnp.float32,
    )
    out_ref[...] = acc / _T


def _chunk_call(x, memt_chunk):
    m, k = x.shape
    return pl.pallas_call(
        _matmul_kernel,
        grid=(_CW // _BN,),
        in_specs=[
            pl.BlockSpec((m, k), lambda i: (0, 0)),
            pl.BlockSpec((k, _BN), lambda i: (0, i)),
        ],
        out_specs=pl.BlockSpec((m, _BN), lambda i: (0, i)),
        out_shape=jax.ShapeDtypeStruct((m, _CW), jnp.float32),
        compiler_params=pltpu.CompilerParams(
            dimension_semantics=("arbitrary",),
            vmem_limit_bytes=63 * 1024 * 1024,
        ),
    )(x, memt_chunk)


@jax.jit
def kernel(x, memory):
    n = memory.shape[0]
    memt = jnp.pad(memory.T, ((0, 0), (0, _NPAD - n)))
    c0 = _chunk_call(x, jax.lax.slice_in_dim(memt, 0, _CW, axis=1))
    c1 = _chunk_call(x, jax.lax.slice_in_dim(memt, _CW, _NPAD, axis=1))
    return jnp.concatenate([c0, c1[:, : n - _CW]], axis=1)


# speedup vs baseline: 1.6182x; 1.6182x over previous
import jax
import jax.numpy as jnp
from jax.experimental import pallas as pl
from jax.experimental.pallas import tpu as pltpu

_T = 0.05
_BN = 4352
_NPAD = 100096


def _matmul_kernel(x_ref, memt_ref, out_ref):
    acc = jax.lax.dot_general(
        x_ref[...],
        memt_ref[...],
        dimension_numbers=(((1,), (0,)), ((), ())),
        preferred_element_type=jnp.float32,
    )
    out_ref[...] = acc / _T


@jax.jit
def kernel(x, memory):
    m, k = x.shape
    n = memory.shape[0]
    memt = memory.T
    grid = (_NPAD // _BN,)
    out = pl.pallas_call(
        _matmul_kernel,
        grid=grid,
        in_specs=[
            pl.BlockSpec((m, k), lambda i: (0, 0)),
            pl.BlockSpec((k, _BN), lambda i: (0, i)),
        ],
        out_specs=pl.BlockSpec((m, _BN), lambda i: (0, i)),
        out_shape=jax.ShapeDtypeStruct((m, _NPAD), jnp.float32),
        compiler_params=pltpu.CompilerParams(
            dimension_semantics=("arbitrary",),
            vmem_limit_bytes=63 * 1024 * 1024,
        ),
    )(x, memt)
    return out[:, :n]
